# baseline (speedup 1.0000x reference)
import functools

import jax
import jax.numpy as jnp
from jax import lax
from jax.experimental import pallas as pl
from jax.experimental.pallas import tpu as pltpu
from jax.experimental.pallas import tpu_sc as plsc

VOCAB = 1000000
EMBED = 64
B = 16384
L = 50

NC = 2
NS = 16
NW = NC * NS
BPW = B // NW
LP = 64
LG = 56
CH = 32
NCHUNK = BPW // CH

_sc_mesh = plsc.VectorSubcoreMesh(core_axis_name="c", subcore_axis_name="s")


@functools.partial(
    pl.kernel,
    mesh=_sc_mesh,
    out_type=jax.ShapeDtypeStruct((B, EMBED), jnp.float32),
    scratch_types=[
        pltpu.VMEM((CH, LP), jnp.int32),
        pltpu.VMEM((CH * LG, EMBED), jnp.float32),
        pltpu.VMEM((CH, EMBED), jnp.float32),
        pltpu.SemaphoreType.DMA,
    ],
    compiler_params=pltpu.CompilerParams(use_tc_tiling_on_sc=False),
)
def _sc_pool(ids_hbm, table_hbm, out_hbm, ids_v, rows_v, pooled_v, sem):
    wid = lax.axis_index("s") * NC + lax.axis_index("c")

    def chunk_body(ci, carry):
        chunk = wid * NCHUNK + ci
        pltpu.sync_copy(ids_hbm.at[pl.ds(chunk * CH, CH)], ids_v)
        descs = []
        for bi in range(CH):
            descs.append(pltpu.async_copy(
                table_hbm.at[ids_v.at[bi, pl.ds(0, LG)]],
                rows_v.at[pl.ds(bi * LG, LG)],
                sem,
            ))
        for d in descs:
            d.wait()

        def b_body(bi, c2):
            row0 = bi * LG
            for col in range(EMBED // 16):
                acc = rows_v[row0, pl.ds(col * 16, 16)]
                for j in range(1, L):
                    acc = acc + rows_v[row0 + j, pl.ds(col * 16, 16)]
                pooled_v[bi, pl.ds(col * 16, 16)] = acc
            return c2

        lax.fori_loop(0, CH, b_body, 0, unroll=False)
        pltpu.sync_copy(pooled_v, out_hbm.at[pl.ds(chunk * CH, CH)])
        return carry

    lax.fori_loop(0, NCHUNK, chunk_body, 0, unroll=False)


def _tc_proj(x_ref, w_ref, b_ref, o_ref):
    x = x_ref[...] * (1.0 / L)
    y = jnp.dot(x, w_ref[...].T, preferred_element_type=jnp.float32)
    y = y + b_ref[...]
    n = jnp.sqrt(jnp.sum(y * y, axis=-1, keepdims=True))
    o_ref[...] = y / jnp.maximum(n, 1e-12)


def kernel(input_ids, table, W, b):
    ids_p = jnp.pad(input_ids, ((0, 0), (0, LP - L)))
    pooled = _sc_pool(ids_p, table)
    out = pl.pallas_call(
        _tc_proj,
        out_shape=jax.ShapeDtypeStruct((B, EMBED), jnp.float32),
    )(pooled, W, b.reshape(1, EMBED))
    return out


# speedup vs baseline: 1.0033x; 1.0033x over previous
import functools

import jax
import jax.numpy as jnp
from jax import lax
from jax.experimental import pallas as pl
from jax.experimental.pallas import tpu as pltpu
from jax.experimental.pallas import tpu_sc as plsc

VOCAB = 1000000
EMBED = 64
B = 16384
L = 50

NC = 2
NS = 16
NW = NC * NS
BPW = B // NW
LP = 64
LG = 56
CH = 16
NCHUNK = BPW // CH

_sc_mesh = plsc.VectorSubcoreMesh(core_axis_name="c", subcore_axis_name="s")


@functools.partial(
    pl.kernel,
    mesh=_sc_mesh,
    out_type=jax.ShapeDtypeStruct((B, EMBED), jnp.float32),
    scratch_types=[
        pltpu.VMEM((CH, LG), jnp.int32),
        pltpu.VMEM((CH * LG, EMBED), jnp.float32),
        pltpu.VMEM((CH, EMBED), jnp.float32),
        pltpu.SemaphoreType.DMA,
    ],
    compiler_params=pltpu.CompilerParams(use_tc_tiling_on_sc=False),
)
def _sc_pool(ids_hbm, table_hbm, out_hbm, ids_v, rows_v, pooled_v, sem):
    wid = lax.axis_index("s") * NC + lax.axis_index("c")

    def chunk_body(ci, carry):
        chunk = wid * NCHUNK + ci
        pltpu.sync_copy(
            ids_hbm.at[pl.ds(chunk * CH, CH), pl.ds(0, LG)], ids_v)
        descs = []
        for bi in range(CH):
            descs.append(pltpu.async_copy(
                table_hbm.at[ids_v.at[bi]],
                rows_v.at[pl.ds(bi * LG, LG)],
                sem,
            ))
        for d in descs:
            d.wait()

        def b_body(bi, c2):
            row0 = bi * LG
            for col in range(EMBED // 16):
                acc = rows_v[row0, pl.ds(col * 16, 16)]
                for j in range(1, L):
                    acc = acc + rows_v[row0 + j, pl.ds(col * 16, 16)]
                pooled_v[bi, pl.ds(col * 16, 16)] = acc
            return c2

        lax.fori_loop(0, CH, b_body, 0, unroll=False)
        pltpu.sync_copy(pooled_v, out_hbm.at[pl.ds(chunk * CH, CH)])
        return carry

    lax.fori_loop(0, NCHUNK, chunk_body, 0, unroll=False)


def _tc_proj(x_ref, w_ref, b_ref, o_ref):
    x = x_ref[...] * (1.0 / L)
    y = jnp.dot(x, w_ref[...].T, preferred_element_type=jnp.float32)
    y = y + b_ref[...]
    n = jnp.sqrt(jnp.sum(y * y, axis=-1, keepdims=True))
    o_ref[...] = y / jnp.maximum(n, 1e-12)


def kernel(input_ids, table, W, b):
    ids_p = jnp.pad(input_ids, ((0, 0), (0, LP - L)))
    pooled = _sc_pool(ids_p, table)
    out = pl.pallas_call(
        _tc_proj,
        out_shape=jax.ShapeDtypeStruct((B, EMBED), jnp.float32),
    )(pooled, W, b.reshape(1, EMBED))
    return out


# speedup vs baseline: 1.0121x; 1.0088x over previous
import functools

import jax
import jax.numpy as jnp
from jax import lax
from jax.experimental import pallas as pl
from jax.experimental.pallas import tpu as pltpu
from jax.experimental.pallas import tpu_sc as plsc

VOCAB = 1000000
EMBED = 64
B = 16384
L = 50

NC = 2
NS = 16
NW = NC * NS
BPW = B // NW
LP = 128
LG = 56
CH = 16
NCHUNK = BPW // CH

_sc_mesh = plsc.VectorSubcoreMesh(core_axis_name="c", subcore_axis_name="s")


@functools.partial(
    pl.kernel,
    mesh=_sc_mesh,
    out_type=jax.ShapeDtypeStruct((B, EMBED), jnp.float32),
    scratch_types=[
        pltpu.VMEM((CH, LP), jnp.int32),
        pltpu.VMEM((CH * LG, EMBED), jnp.float32),
        pltpu.VMEM((CH, EMBED), jnp.float32),
        pltpu.SemaphoreType.DMA,
    ],
    compiler_params=pltpu.CompilerParams(use_tc_tiling_on_sc=False),
)
def _sc_pool(ids_hbm, table_hbm, out_hbm, ids_v, rows_v, pooled_v, sem):
    wid = lax.axis_index("s") * NC + lax.axis_index("c")

    def chunk_body(ci, carry):
        chunk = wid * NCHUNK + ci
        pltpu.sync_copy(ids_hbm.at[pl.ds(chunk * CH, CH)], ids_v)
        descs = []
        for bi in range(CH):
            descs.append(pltpu.async_copy(
                table_hbm.at[ids_v.at[bi, pl.ds(0, LG)]],
                rows_v.at[pl.ds(bi * LG, LG)],
                sem,
            ))
        for d in descs:
            d.wait()

        def b_body(bi, c2):
            row0 = bi * LG
            for col in range(EMBED // 16):
                acc = rows_v[row0, pl.ds(col * 16, 16)]
                for j in range(1, L):
                    acc = acc + rows_v[row0 + j, pl.ds(col * 16, 16)]
                pooled_v[bi, pl.ds(col * 16, 16)] = acc
            return c2

        lax.fori_loop(0, CH, b_body, 0, unroll=False)
        pltpu.sync_copy(pooled_v, out_hbm.at[pl.ds(chunk * CH, CH)])
        return carry

    lax.fori_loop(0, NCHUNK, chunk_body, 0, unroll=False)


def _tc_proj(x_ref, w_ref, b_ref, o_ref):
    x = x_ref[...] * (1.0 / L)
    y = jnp.dot(x, w_ref[...].T, preferred_element_type=jnp.float32)
    y = y + b_ref[...]
    n = jnp.sqrt(jnp.sum(y * y, axis=-1, keepdims=True))
    o_ref[...] = y / jnp.maximum(n, 1e-12)


def kernel(input_ids, table, W, b):
    ids_p = jnp.pad(input_ids, ((0, 0), (0, LP - L)))
    pooled = _sc_pool(ids_p, table)
    out = pl.pallas_call(
        _tc_proj,
        out_shape=jax.ShapeDtypeStruct((B, EMBED), jnp.float32),
    )(pooled, W, b.reshape(1, EMBED))
    return out


# speedup vs baseline: 3.3539x; 3.3137x over previous
import functools

import jax
import jax.numpy as jnp
from jax import lax
from jax.experimental import pallas as pl
from jax.experimental.pallas import tpu as pltpu
from jax.experimental.pallas import tpu_sc as plsc

VOCAB = 1000000
EMBED = 64
B = 16384
L = 50

NC = 2
NS = 16
NW = NC * NS
BPW = B // NW
CB = 32
NCHUNK = BPW // CB

_sc_mesh = plsc.VectorSubcoreMesh(core_axis_name="c", subcore_axis_name="s")


@functools.partial(
    pl.kernel,
    mesh=_sc_mesh,
    out_type=jax.ShapeDtypeStruct((B, EMBED), jnp.float32),
    scratch_types=[
        pltpu.VMEM((L, CB), jnp.int32),
        pltpu.VMEM((L * CB, EMBED), jnp.float32),
        pltpu.VMEM((CB, EMBED), jnp.float32),
        pltpu.SemaphoreType.DMA,
    ],
    compiler_params=pltpu.CompilerParams(use_tc_tiling_on_sc=False),
)
def _sc_pool(ids_hbm, table_hbm, out_hbm, ids_v, rows_v, pooled_v, sem):
    wid = lax.axis_index("s") * NC + lax.axis_index("c")

    def chunk_body(ci, carry):
        base = wid * BPW + ci * CB
        pltpu.sync_copy(ids_hbm.at[pl.ds(0, L), pl.ds(base, CB)], ids_v)
        descs = []
        for l in range(L):
            descs.append(pltpu.async_copy(
                table_hbm.at[ids_v.at[l]],
                rows_v.at[pl.ds(l * CB, CB)],
                sem,
            ))
        for d in descs:
            d.wait()

        def b_body(bi, c2):
            for col in range(EMBED // 16):
                acc = rows_v[bi, pl.ds(col * 16, 16)]
                for l in range(1, L):
                    acc = acc + rows_v[l * CB + bi, pl.ds(col * 16, 16)]
                pooled_v[bi, pl.ds(col * 16, 16)] = acc
            return c2

        lax.fori_loop(0, CB, b_body, 0, unroll=False)
        pltpu.sync_copy(pooled_v, out_hbm.at[pl.ds(base, CB)])
        return carry

    lax.fori_loop(0, NCHUNK, chunk_body, 0, unroll=False)


def _tc_proj(x_ref, w_ref, b_ref, o_ref):
    x = x_ref[...] * (1.0 / L)
    y = jnp.dot(x, w_ref[...].T, preferred_element_type=jnp.float32)
    y = y + b_ref[...]
    n = jnp.sqrt(jnp.sum(y * y, axis=-1, keepdims=True))
    o_ref[...] = y / jnp.maximum(n, 1e-12)


def kernel(input_ids, table, W, b):
    pooled = _sc_pool(input_ids.T, table)
    out = pl.pallas_call(
        _tc_proj,
        out_shape=jax.ShapeDtypeStruct((B, EMBED), jnp.float32),
    )(pooled, W, b.reshape(1, EMBED))
    return out


# speedup vs baseline: 3.3592x; 1.0016x over previous
import functools

import jax
import jax.numpy as jnp
from jax import lax
from jax.experimental import pallas as pl
from jax.experimental.pallas import tpu as pltpu
from jax.experimental.pallas import tpu_sc as plsc

VOCAB = 1000000
EMBED = 64
B = 16384
L = 50

NC = 2
NS = 16
NW = NC * NS
BPW = B // NW
CB = 32
NCHUNK = BPW // CB

_sc_mesh = plsc.VectorSubcoreMesh(core_axis_name="c", subcore_axis_name="s")


@functools.partial(
    pl.kernel,
    mesh=_sc_mesh,
    out_type=jax.ShapeDtypeStruct((B, EMBED), jnp.float32),
    scratch_types=[
        pltpu.VMEM((L, CB), jnp.int32),
        pltpu.VMEM((L * CB, EMBED), jnp.float32),
        pltpu.VMEM((CB, EMBED), jnp.float32),
        pltpu.SemaphoreType.DMA,
    ],
    compiler_params=pltpu.CompilerParams(use_tc_tiling_on_sc=False),
)
def _sc_pool(ids_hbm, table_hbm, out_hbm, ids_v, rows_v, pooled_v, sem):
    wid = lax.axis_index("s") * NC + lax.axis_index("c")

    def chunk_body(ci, carry):
        base = wid * BPW + ci * CB
        pltpu.sync_copy(ids_hbm.at[pl.ds(0, L), pl.ds(base, CB)], ids_v)
        descs = []
        for l in range(L):
            descs.append(pltpu.async_copy(
                table_hbm.at[ids_v.at[l]],
                rows_v.at[pl.ds(l * CB, CB)],
                sem,
            ))
        for d in descs:
            d.wait()

        def b_body(bi, c2):
            for col in range(EMBED // 16):
                acc = rows_v[bi, pl.ds(col * 16, 16)]
                for l in range(1, L):
                    acc = acc + rows_v[l * CB + bi, pl.ds(col * 16, 16)]
                pooled_v[bi, pl.ds(col * 16, 16)] = acc
            return c2

        lax.fori_loop(0, CB, b_body, 0, unroll=False)
        pltpu.sync_copy(pooled_v, out_hbm.at[pl.ds(base, CB)])
        return carry

    lax.fori_loop(0, NCHUNK, chunk_body, 0, unroll=False)


def _tc_transpose_ids(x_ref, o_ref):
    o_ref[...] = x_ref[...].T


def _tc_proj(x_ref, w_ref, b_ref, o_ref):
    x = x_ref[...] * (1.0 / L)
    y = jnp.dot(x, w_ref[...].T, preferred_element_type=jnp.float32)
    y = y + b_ref[...]
    n = jnp.sqrt(jnp.sum(y * y, axis=-1, keepdims=True))
    o_ref[...] = y / jnp.maximum(n, 1e-12)


def kernel(input_ids, table, W, b):
    ids_t = pl.pallas_call(
        _tc_transpose_ids,
        grid=(32,),
        in_specs=[pl.BlockSpec((B // 32, L), lambda i: (i, 0))],
        out_specs=pl.BlockSpec((L, B // 32), lambda i: (0, i)),
        out_shape=jax.ShapeDtypeStruct((L, B), jnp.int32),
    )(input_ids)
    pooled = _sc_pool(ids_t, table)
    out = pl.pallas_call(
        _tc_proj,
        out_shape=jax.ShapeDtypeStruct((B, EMBED), jnp.float32),
    )(pooled, W, b.reshape(1, EMBED))
    return out


# speedup vs baseline: 3.3637x; 1.0013x over previous
import functools

import jax
import jax.numpy as jnp
from jax import lax
from jax.experimental import pallas as pl
from jax.experimental.pallas import tpu as pltpu
from jax.experimental.pallas import tpu_sc as plsc

VOCAB = 1000000
EMBED = 64
B = 16384
L = 50

NC = 2
NS = 16
NW = NC * NS
BPW = B // NW
CB = 32
NCHUNK = BPW // CB

_sc_mesh = plsc.VectorSubcoreMesh(core_axis_name="c", subcore_axis_name="s")


@functools.partial(
    pl.kernel,
    mesh=_sc_mesh,
    out_type=jax.ShapeDtypeStruct((B, EMBED), jnp.float32),
    scratch_types=[
        pltpu.VMEM((L, CB), jnp.int32),
        pltpu.VMEM((L * CB, EMBED), jnp.float32),
        pltpu.VMEM((CB, EMBED), jnp.float32),
        pltpu.SemaphoreType.DMA,
    ],
    compiler_params=pltpu.CompilerParams(use_tc_tiling_on_sc=False),
)
def _sc_pool(ids_hbm, table_hbm, out_hbm, ids_v, rows_v, pooled_v, sem):
    wid = lax.axis_index("s") * NC + lax.axis_index("c")

    def chunk_body(ci, carry):
        base = wid * BPW + ci * CB
        pltpu.sync_copy(
            ids_hbm.at[pl.ds(0, L), base // 128, pl.ds(base % 128, CB)],
            ids_v)
        descs = []
        for l in range(L):
            descs.append(pltpu.async_copy(
                table_hbm.at[ids_v.at[l]],
                rows_v.at[pl.ds(l * CB, CB)],
                sem,
            ))
        for d in descs:
            d.wait()

        def b_body(bi, c2):
            for col in range(EMBED // 16):
                acc = rows_v[bi, pl.ds(col * 16, 16)]
                for l in range(1, L):
                    acc = acc + rows_v[l * CB + bi, pl.ds(col * 16, 16)]
                pooled_v[bi, pl.ds(col * 16, 16)] = acc
            return c2

        lax.fori_loop(0, CB, b_body, 0, unroll=False)
        pltpu.sync_copy(pooled_v, out_hbm.at[pl.ds(base, CB)])
        return carry

    lax.fori_loop(0, NCHUNK, chunk_body, 0, unroll=False)


def _tc_transpose_ids(x_ref, o_ref):
    o_ref[...] = x_ref[...].T.reshape(L, 8, 128)


def _tc_proj(x_ref, w_ref, b_ref, o_ref):
    x = x_ref[...] * (1.0 / L)
    y = jnp.dot(x, w_ref[...].T, preferred_element_type=jnp.float32)
    y = y + b_ref[...]
    n = jnp.sqrt(jnp.sum(y * y, axis=-1, keepdims=True))
    o_ref[...] = y / jnp.maximum(n, 1e-12)


def kernel(input_ids, table, W, b):
    ids_t = pl.pallas_call(
        _tc_transpose_ids,
        grid=(16,),
        in_specs=[pl.BlockSpec((B // 16, L), lambda i: (i, 0))],
        out_specs=pl.BlockSpec((L, 8, 128), lambda i: (0, i, 0)),
        out_shape=jax.ShapeDtypeStruct((L, B // 128, 128), jnp.int32),
    )(input_ids)
    pooled = _sc_pool(ids_t, table)
    out = pl.pallas_call(
        _tc_proj,
        out_shape=jax.ShapeDtypeStruct((B, EMBED), jnp.float32),
    )(pooled, W, b.reshape(1, EMBED))
    return out
